# Initial kernel scaffold; baseline (speedup 1.0000x reference)
import jax
import jax.numpy as jnp
from jax.experimental import pallas as pl


def kernel(prior, t):
    raise NotImplementedError("write your pallas kernel here")


# speedup vs baseline: 100.4342x; 100.4342x over previous
import functools

import jax
import jax.numpy as jnp
from jax import lax
from jax.experimental import pallas as pl
from jax.experimental.pallas import tpu as pltpu
from jax.experimental.pallas import tpu_sc as plsc

_L = 16
_NB = 1024


def _iota():
    return lax.iota(jnp.int32, _L)


def _scalar(x):
    return jnp.max(x)


def _popcount(m):
    return _scalar(plsc.all_reduce_population_count(m))


def _hist_select(hist_ref, target, total):
    target_asc = total - target + 1

    def scan_vregs(j, carry):
        cum, found, jc, cum_before = carry
        s = jnp.sum(hist_ref[pl.ds(j * _L, _L)])
        crossing = jnp.logical_and(found == 0, cum + s >= target_asc)
        jc = jnp.where(crossing, j, jc)
        cum_before = jnp.where(crossing, cum, cum_before)
        found = jnp.where(crossing, jnp.int32(1), found)
        return cum + s, found, jc, cum_before

    zero = jnp.int32(0)
    _, _, jc, cum_before = lax.fori_loop(
        0, _NB // _L, scan_vregs, (zero, zero, zero, zero))
    h = hist_ref[pl.ds(jc * _L, _L)]
    pref = plsc.cumsum(h)
    lane = _scalar(plsc.all_reduce_ffs(cum_before + pref >= target_asc))
    bucket = jc * _L + lane
    cum_at_b = cum_before + jnp.sum(jnp.where(_iota() <= lane, h, 0))
    rem = target - (total - cum_at_b)
    return bucket, rem


def _sc_topk_mask(prior_bits, ks):
    n_rows, s = prior_bits.shape
    info = plsc.get_sparse_core_info()
    nw = info.num_cores * info.num_subcores
    rows_per_w = n_rows // nw
    nv_row = s // _L
    mesh = plsc.VectorSubcoreMesh(core_axis_name="c", subcore_axis_name="s")

    @functools.partial(
        pl.kernel,
        mesh=mesh,
        out_type=jax.ShapeDtypeStruct((n_rows, s), jnp.int32),
        scratch_types=[
            pltpu.VMEM((s,), jnp.int32),
            pltpu.VMEM((s,), jnp.int32),
            pltpu.VMEM((s + _L,), jnp.int32),
            pltpu.VMEM((_NB,), jnp.int32),
            pltpu.VMEM((n_rows + _L,), jnp.int32),
        ],
        compiler_params=pltpu.CompilerParams(needs_layout_passes=False),
    )
    def body(prior_hbm, ks_hbm, out_hbm, row_v, mask_v, cand_v, hist_v, ks_v):
        wid = lax.axis_index("s") * info.num_cores + lax.axis_index("c")
        pltpu.sync_copy(ks_hbm, ks_v.at[pl.ds(0, n_rows)])
        ones = jnp.ones((_L,), jnp.int32)

        def zero_hist(j, _):
            hist_v[pl.ds(j * _L, _L)] = jnp.zeros((_L,), jnp.int32)
            return jnp.int32(0)

        def do_row(i, _):
            r = wid * rows_per_w + i
            pltpu.sync_copy(prior_hbm.at[r], row_v)
            k = ks_v[pl.ds(r, _L)][0]
            kk = jnp.maximum(k, 1)

            lax.fori_loop(0, _NB // _L, zero_hist, jnp.int32(0))

            def p1(j, _):
                u = row_v[pl.ds(j * _L, _L)]
                plsc.addupdate_scatter(hist_v, [u >> 20], ones)
                return jnp.int32(0)

            lax.fori_loop(0, nv_row, p1, jnp.int32(0))
            b1, rem1 = _hist_select(hist_v, kk, jnp.int32(s))

            def p2(j, off):
                u = row_v[pl.ds(j * _L, _L)]
                b = u >> 20
                mask_v[pl.ds(j * _L, _L)] = jnp.where(b > b1, 1, 0)
                meq = b == b1
                plsc.store_compressed(
                    cand_v.at[pl.ds(off, _L)], j * _L + _iota(), mask=meq)
                return off + _popcount(meq)

            n1 = lax.fori_loop(0, nv_row, p2, jnp.int32(0))

            def refine(shift, ncand, target):
                lax.fori_loop(0, _NB // _L, zero_hist, jnp.int32(0))
                nv = (ncand + _L - 1) >> 4

                def hloop(j, _):
                    lm = j * _L + _iota() < ncand
                    idx = jnp.where(lm, cand_v[pl.ds(j * _L, _L)], 0)
                    u = plsc.load_gather(row_v, [idx], mask=lm)
                    bb = (u >> shift) & (_NB - 1)
                    plsc.addupdate_scatter(hist_v, [bb], ones, mask=lm)
                    return jnp.int32(0)

                lax.fori_loop(0, nv, hloop, jnp.int32(0))
                bx, remx = _hist_select(hist_v, target, ncand)

                def cloop(j, off):
                    lm = j * _L + _iota() < ncand
                    idx = jnp.where(lm, cand_v[pl.ds(j * _L, _L)], 0)
                    u = plsc.load_gather(row_v, [idx], mask=lm)
                    bb = (u >> shift) & (_NB - 1)
                    gt = jnp.logical_and(lm, bb > bx)
                    plsc.store_scatter(mask_v, [idx], ones, mask=gt)
                    meq = jnp.logical_and(lm, bb == bx)
                    plsc.store_compressed(
                        cand_v.at[pl.ds(off, _L)], idx, mask=meq)
                    return off + _popcount(meq)

                nnew = lax.fori_loop(0, nv, cloop, jnp.int32(0))
                return nnew, remx

            n2, rem2 = refine(10, n1, rem1)
            n3, rem3 = refine(0, n2, rem2)

            need_eq = jnp.where(k == 0, jnp.int32(0), rem3)

            def floop(j, _):
                m = j * _L + _iota() < need_eq
                idx = jnp.where(m, cand_v[pl.ds(j * _L, _L)], 0)
                plsc.store_scatter(mask_v, [idx], ones, mask=m)
                return jnp.int32(0)

            lax.fori_loop(0, (need_eq + _L - 1) >> 4, floop, jnp.int32(0))
            pltpu.sync_copy(mask_v, out_hbm.at[r])
            return jnp.int32(0)

        lax.fori_loop(0, rows_per_w, do_row, jnp.int32(0))

    return body(prior_bits, ks)


def kernel(prior, t):
    rates = 1.0 - jnp.cos(jnp.pi * t / 2.0)
    weights = 0.5 * jnp.pi * jnp.sin(jnp.pi * t / 2.0)
    ks = (prior.shape[-1] * rates).astype(jnp.int32).reshape(-1)
    prior_bits = lax.bitcast_convert_type(prior, jnp.int32)
    mask = _sc_topk_mask(prior_bits, ks).astype(bool)
    return mask, weights


# speedup vs baseline: 129.4236x; 1.2886x over previous
import functools

import jax
import jax.numpy as jnp
from jax import lax
from jax.experimental import pallas as pl
from jax.experimental.pallas import tpu as pltpu
from jax.experimental.pallas import tpu_sc as plsc

_L = 16
_NB = 1024


def _iota():
    return lax.iota(jnp.int32, _L)


def _scalar(x):
    return jnp.max(x)


def _popcount(m):
    return _scalar(plsc.all_reduce_population_count(m))


def _hist_select(hist_ref, target, total):
    target_asc = total - target + 1

    def scan_vregs(j, carry):
        cum, found, jc, cum_before = carry
        s = jnp.sum(hist_ref[pl.ds(j * _L, _L)])
        crossing = jnp.logical_and(found == 0, cum + s >= target_asc)
        jc = jnp.where(crossing, j, jc)
        cum_before = jnp.where(crossing, cum, cum_before)
        found = jnp.where(crossing, jnp.int32(1), found)
        return cum + s, found, jc, cum_before

    zero = jnp.int32(0)
    _, _, jc, cum_before = lax.fori_loop(
        0, _NB // _L, scan_vregs, (zero, zero, zero, zero))
    h = hist_ref[pl.ds(jc * _L, _L)]
    pref = plsc.cumsum(h)
    lane = _scalar(plsc.all_reduce_ffs(cum_before + pref >= target_asc))
    bucket = jc * _L + lane
    cum_at_b = cum_before + jnp.sum(jnp.where(_iota() <= lane, h, 0))
    rem = target - (total - cum_at_b)
    return bucket, rem


def _sc_topk_mask(prior_bits, ks):
    n_rows, s = prior_bits.shape
    info = plsc.get_sparse_core_info()
    nw = info.num_cores * info.num_subcores
    rows_per_w = n_rows // nw
    nv_row = s // _L
    mesh = plsc.VectorSubcoreMesh(core_axis_name="c", subcore_axis_name="s")

    @functools.partial(
        pl.kernel,
        mesh=mesh,
        out_type=jax.ShapeDtypeStruct((n_rows, s), jnp.int32),
        scratch_types=[
            pltpu.VMEM((s,), jnp.int32),
            pltpu.VMEM((s,), jnp.int32),
            pltpu.VMEM((s + _L,), jnp.int32),
            pltpu.VMEM((_NB,), jnp.int32),
            pltpu.VMEM((n_rows + _L,), jnp.int32),
        ],
        compiler_params=pltpu.CompilerParams(needs_layout_passes=False),
    )
    def body(prior_hbm, ks_hbm, out_hbm, row_v, mask_v, cand_v, hist_v, ks_v):
        wid = lax.axis_index("s") * info.num_cores + lax.axis_index("c")
        pltpu.sync_copy(ks_hbm, ks_v.at[pl.ds(0, n_rows)])
        ones = jnp.ones((_L,), jnp.int32)

        def zero_hist():
            zeros = jnp.zeros((_L,), jnp.int32)
            for j in range(_NB // _L):
                hist_v[pl.ds(j * _L, _L)] = zeros

        def do_row(i, _):
            r = wid * rows_per_w + i
            pltpu.sync_copy(prior_hbm.at[r], row_v)
            k = ks_v[pl.ds(r, _L)][0]
            kk = jnp.maximum(k, 1)

            zero_hist()

            @plsc.parallel_loop(0, nv_row, 1, unroll=8)
            def _(j):
                u = row_v[pl.ds(j * _L, _L)]
                plsc.addupdate_scatter(hist_v, [u >> 20], ones)

            b1, rem1 = _hist_select(hist_v, kk, jnp.int32(s))

            def p2(j, off):
                u = row_v[pl.ds(j * _L, _L)]
                b = u >> 20
                mask_v[pl.ds(j * _L, _L)] = jnp.where(b > b1, 1, 0)
                meq = b == b1
                plsc.store_compressed(
                    cand_v.at[pl.ds(off, _L)], j * _L + _iota(), mask=meq)
                return off + _popcount(meq)

            n1 = lax.fori_loop(0, nv_row, p2, jnp.int32(0))

            def refine(shift, ncand, target):
                zero_hist()
                nv = (ncand + _L - 1) >> 4

                def hloop(j, _):
                    lm = j * _L + _iota() < ncand
                    idx = jnp.where(lm, cand_v[pl.ds(j * _L, _L)], 0)
                    u = plsc.load_gather(row_v, [idx], mask=lm)
                    bb = (u >> shift) & (_NB - 1)
                    plsc.addupdate_scatter(hist_v, [bb], ones, mask=lm)
                    return jnp.int32(0)

                lax.fori_loop(0, nv, hloop, jnp.int32(0))
                bx, remx = _hist_select(hist_v, target, ncand)

                def cloop(j, off):
                    lm = j * _L + _iota() < ncand
                    idx = jnp.where(lm, cand_v[pl.ds(j * _L, _L)], 0)
                    u = plsc.load_gather(row_v, [idx], mask=lm)
                    bb = (u >> shift) & (_NB - 1)
                    gt = jnp.logical_and(lm, bb > bx)
                    plsc.store_scatter(mask_v, [idx], ones, mask=gt)
                    meq = jnp.logical_and(lm, bb == bx)
                    plsc.store_compressed(
                        cand_v.at[pl.ds(off, _L)], idx, mask=meq)
                    return off + _popcount(meq)

                nnew = lax.fori_loop(0, nv, cloop, jnp.int32(0))
                return nnew, remx

            n2, rem2 = refine(10, n1, rem1)
            n3, rem3 = refine(0, n2, rem2)

            need_eq = jnp.where(k == 0, jnp.int32(0), rem3)

            def floop(j, _):
                m = j * _L + _iota() < need_eq
                idx = jnp.where(m, cand_v[pl.ds(j * _L, _L)], 0)
                plsc.store_scatter(mask_v, [idx], ones, mask=m)
                return jnp.int32(0)

            lax.fori_loop(0, (need_eq + _L - 1) >> 4, floop, jnp.int32(0))
            pltpu.sync_copy(mask_v, out_hbm.at[r])
            return jnp.int32(0)

        lax.fori_loop(0, rows_per_w, do_row, jnp.int32(0))

    return body(prior_bits, ks)


def kernel(prior, t):
    rates = 1.0 - jnp.cos(jnp.pi * t / 2.0)
    weights = 0.5 * jnp.pi * jnp.sin(jnp.pi * t / 2.0)
    ks = (prior.shape[-1] * rates).astype(jnp.int32).reshape(-1)
    prior_bits = lax.bitcast_convert_type(prior, jnp.int32)
    mask = _sc_topk_mask(prior_bits, ks).astype(bool)
    return mask, weights


# speedup vs baseline: 139.0523x; 1.0744x over previous
import functools

import jax
import jax.numpy as jnp
from jax import lax
from jax.experimental import pallas as pl
from jax.experimental.pallas import tpu as pltpu
from jax.experimental.pallas import tpu_sc as plsc

_L = 16
_NB = 1024


def _iota():
    return lax.iota(jnp.int32, _L)


def _scalar(x):
    return x[0]


def _popcount(m):
    return _scalar(plsc.all_reduce_population_count(m))


def _hist_select(hist_ref, target, total):
    target_asc = total - target + 1

    def scan_vregs(j, carry):
        cum, found, jc, cum_before = carry
        s = jnp.sum(hist_ref[pl.ds(j * _L, _L)])
        crossing = jnp.logical_and(found == 0, cum + s >= target_asc)
        jc = jnp.where(crossing, j, jc)
        cum_before = jnp.where(crossing, cum, cum_before)
        found = jnp.where(crossing, jnp.int32(1), found)
        return cum + s, found, jc, cum_before

    zero = jnp.int32(0)
    _, _, jc, cum_before = lax.fori_loop(
        0, _NB // _L, scan_vregs, (zero, zero, zero, zero), unroll=4)
    h = hist_ref[pl.ds(jc * _L, _L)]
    pref = plsc.cumsum(h)
    lane = _scalar(plsc.all_reduce_ffs(cum_before + pref >= target_asc))
    bucket = jc * _L + lane
    cum_at_b = cum_before + jnp.sum(jnp.where(_iota() <= lane, h, 0))
    rem = target - (total - cum_at_b)
    return bucket, rem


def _sc_topk_mask(prior_bits, ks):
    n_rows, s = prior_bits.shape
    info = plsc.get_sparse_core_info()
    nw = info.num_cores * info.num_subcores
    rows_per_w = n_rows // nw
    nv_row = s // _L
    mesh = plsc.VectorSubcoreMesh(core_axis_name="c", subcore_axis_name="s")

    @functools.partial(
        pl.kernel,
        mesh=mesh,
        out_type=jax.ShapeDtypeStruct((n_rows, s), jnp.int32),
        scratch_types=[
            pltpu.VMEM((s,), jnp.int32),
            pltpu.VMEM((s,), jnp.int32),
            pltpu.VMEM((s + _L,), jnp.int32),
            pltpu.VMEM((_NB,), jnp.int32),
            pltpu.VMEM((n_rows + _L,), jnp.int32),
        ],
        compiler_params=pltpu.CompilerParams(needs_layout_passes=False),
    )
    def body(prior_hbm, ks_hbm, out_hbm, row_v, mask_v, cand_v, hist_v, ks_v):
        wid = lax.axis_index("s") * info.num_cores + lax.axis_index("c")
        pltpu.sync_copy(ks_hbm, ks_v.at[pl.ds(0, n_rows)])
        ones = jnp.ones((_L,), jnp.int32)

        def zero_hist():
            zeros = jnp.zeros((_L,), jnp.int32)
            for j in range(_NB // _L):
                hist_v[pl.ds(j * _L, _L)] = zeros

        def do_row(i, _):
            r = wid * rows_per_w + i
            pltpu.sync_copy(prior_hbm.at[r], row_v)
            k = ks_v[pl.ds(r, _L)][0]
            kk = jnp.maximum(k, 1)

            zero_hist()

            @plsc.parallel_loop(0, nv_row, 1, unroll=8)
            def _(j):
                u = row_v[pl.ds(j * _L, _L)]
                plsc.addupdate_scatter(hist_v, [u >> 20], ones)

            b1, rem1 = _hist_select(hist_v, kk, jnp.int32(s))

            def p2(j, off):
                u = row_v[pl.ds(j * _L, _L)]
                b = u >> 20
                mask_v[pl.ds(j * _L, _L)] = jnp.where(b > b1, 1, 0)
                meq = b == b1
                plsc.store_compressed(
                    cand_v.at[pl.ds(off, _L)], j * _L + _iota(), mask=meq)
                return off + _popcount(meq)

            n1 = lax.fori_loop(0, nv_row, p2, jnp.int32(0), unroll=4)

            def refine(shift, ncand, target):
                zero_hist()
                nv = (ncand + _L - 1) >> 4

                def hloop(j, _):
                    lm = j * _L + _iota() < ncand
                    idx = jnp.where(lm, cand_v[pl.ds(j * _L, _L)], 0)
                    u = plsc.load_gather(row_v, [idx], mask=lm)
                    bb = (u >> shift) & (_NB - 1)
                    plsc.addupdate_scatter(hist_v, [bb], ones, mask=lm)
                    return jnp.int32(0)

                lax.fori_loop(0, nv, hloop, jnp.int32(0))
                bx, remx = _hist_select(hist_v, target, ncand)

                def cloop(j, off):
                    lm = j * _L + _iota() < ncand
                    idx = jnp.where(lm, cand_v[pl.ds(j * _L, _L)], 0)
                    u = plsc.load_gather(row_v, [idx], mask=lm)
                    bb = (u >> shift) & (_NB - 1)
                    gt = jnp.logical_and(lm, bb > bx)
                    plsc.store_scatter(mask_v, [idx], ones, mask=gt)
                    meq = jnp.logical_and(lm, bb == bx)
                    plsc.store_compressed(
                        cand_v.at[pl.ds(off, _L)], idx, mask=meq)
                    return off + _popcount(meq)

                nnew = lax.fori_loop(0, nv, cloop, jnp.int32(0))
                return nnew, remx

            n2, rem2 = refine(10, n1, rem1)
            n3, rem3 = refine(0, n2, rem2)

            need_eq = jnp.where(k == 0, jnp.int32(0), rem3)

            def floop(j, _):
                m = j * _L + _iota() < need_eq
                idx = jnp.where(m, cand_v[pl.ds(j * _L, _L)], 0)
                plsc.store_scatter(mask_v, [idx], ones, mask=m)
                return jnp.int32(0)

            lax.fori_loop(0, (need_eq + _L - 1) >> 4, floop, jnp.int32(0))
            pltpu.sync_copy(mask_v, out_hbm.at[r])
            return jnp.int32(0)

        lax.fori_loop(0, rows_per_w, do_row, jnp.int32(0))

    return body(prior_bits, ks)


def kernel(prior, t):
    rates = 1.0 - jnp.cos(jnp.pi * t / 2.0)
    weights = 0.5 * jnp.pi * jnp.sin(jnp.pi * t / 2.0)
    ks = (prior.shape[-1] * rates).astype(jnp.int32).reshape(-1)
    prior_bits = lax.bitcast_convert_type(prior, jnp.int32)
    mask = _sc_topk_mask(prior_bits, ks).astype(bool)
    return mask, weights


# speedup vs baseline: 195.6861x; 1.4073x over previous
import functools

import jax
import jax.numpy as jnp
from jax import lax
from jax.experimental import pallas as pl
from jax.experimental.pallas import tpu as pltpu
from jax.experimental.pallas import tpu_sc as plsc

_L = 16
_NB = 1024


def _iota():
    return lax.iota(jnp.int32, _L)


def _scalar(x):
    return x[0]


def _popcount(m):
    return _scalar(plsc.all_reduce_population_count(m))


def _hist_select(hist_ref, target, total):
    target_asc = total - target + 1

    def scan_vregs(j, carry):
        cum, found, jc, cum_before = carry
        s = jnp.sum(hist_ref[pl.ds(j * _L, _L)])
        crossing = jnp.logical_and(found == 0, cum + s >= target_asc)
        jc = jnp.where(crossing, j, jc)
        cum_before = jnp.where(crossing, cum, cum_before)
        found = jnp.where(crossing, jnp.int32(1), found)
        return cum + s, found, jc, cum_before

    zero = jnp.int32(0)
    _, _, jc, cum_before = lax.fori_loop(
        0, _NB // _L, scan_vregs, (zero, zero, zero, zero), unroll=4)
    h = hist_ref[pl.ds(jc * _L, _L)]
    pref = plsc.cumsum(h)
    lane = _scalar(plsc.all_reduce_ffs(cum_before + pref >= target_asc))
    bucket = jc * _L + lane
    cum_at_b = cum_before + jnp.sum(jnp.where(_iota() <= lane, h, 0))
    rem = target - (total - cum_at_b)
    return bucket, rem


def _sc_topk_mask(prior_bits, ks):
    n_rows, s = prior_bits.shape
    info = plsc.get_sparse_core_info()
    nw = info.num_cores * info.num_subcores
    rows_per_w = n_rows // nw
    nv_row = s // _L
    mesh = plsc.VectorSubcoreMesh(core_axis_name="c", subcore_axis_name="s")

    @functools.partial(
        pl.kernel,
        mesh=mesh,
        out_type=jax.ShapeDtypeStruct((n_rows, s), jnp.int32),
        scratch_types=[
            pltpu.VMEM((s,), jnp.int32),
            pltpu.VMEM((s,), jnp.int32),
            pltpu.VMEM((_NB + _L,), jnp.int32),
            pltpu.VMEM((n_rows + _L,), jnp.int32),
        ],
        compiler_params=pltpu.CompilerParams(needs_layout_passes=False),
    )
    def body(prior_hbm, ks_hbm, out_hbm, row_v, mask_v, hist_v, ks_v):
        wid = lax.axis_index("s") * info.num_cores + lax.axis_index("c")
        pltpu.sync_copy(ks_hbm, ks_v.at[pl.ds(0, n_rows)])
        ones = jnp.ones((_L,), jnp.int32)

        def zero_hist():
            zeros = jnp.zeros((_L,), jnp.int32)
            for j in range(_NB // _L):
                hist_v[pl.ds(j * _L, _L)] = zeros

        def hist_at(b):
            return _scalar(hist_v[pl.ds(b, _L)])

        def do_row(i, _):
            r = wid * rows_per_w + i
            pltpu.sync_copy(prior_hbm.at[r], row_v)
            k = ks_v[pl.ds(r, _L)][0]
            kk = jnp.maximum(k, 1)

            zero_hist()

            @plsc.parallel_loop(0, nv_row, 1, unroll=8)
            def _(j):
                u = row_v[pl.ds(j * _L, _L)]
                plsc.addupdate_scatter(hist_v, [u >> 20], ones)

            b1, rem1 = _hist_select(hist_v, kk, jnp.int32(s))
            n1 = hist_at(b1)

            zero_hist()

            @plsc.parallel_loop(0, nv_row, 1, unroll=8)
            def _(j):
                u = row_v[pl.ds(j * _L, _L)]
                m2 = (u >> 20) == b1
                plsc.addupdate_scatter(
                    hist_v, [(u >> 10) & (_NB - 1)], ones, mask=m2)

            b2, rem2 = _hist_select(hist_v, rem1, n1)
            n2 = hist_at(b2)
            hi20 = (b1 << 10) | b2

            zero_hist()

            @plsc.parallel_loop(0, nv_row, 1, unroll=8)
            def _(j):
                u = row_v[pl.ds(j * _L, _L)]
                m3 = (u >> 10) == hi20
                plsc.addupdate_scatter(hist_v, [u & (_NB - 1)], ones, mask=m3)

            b3, rem3 = _hist_select(hist_v, rem2, n2)
            n3 = hist_at(b3)
            thresh = (hi20 << 10) | b3

            def mask_zero():
                @plsc.parallel_loop(0, nv_row, 1, unroll=8)
                def _(j):
                    mask_v[pl.ds(j * _L, _L)] = jnp.zeros((_L,), jnp.int32)
                return jnp.int32(0)

            def mask_ge():
                @plsc.parallel_loop(0, nv_row, 1, unroll=8)
                def _(j):
                    u = row_v[pl.ds(j * _L, _L)]
                    mask_v[pl.ds(j * _L, _L)] = jnp.where(u >= thresh, 1, 0)
                return jnp.int32(0)

            def mask_tie():
                def tie(j, cnt):
                    u = row_v[pl.ds(j * _L, _L)]
                    gt = u > thresh
                    eq = u == thresh
                    pref = plsc.cumsum(jnp.where(eq, 1, 0))
                    sel = jnp.logical_and(eq, cnt + pref <= rem3)
                    mask_v[pl.ds(j * _L, _L)] = jnp.where(
                        jnp.logical_or(gt, sel), 1, 0)
                    return cnt + pref[_L - 1]

                lax.fori_loop(0, nv_row, tie, jnp.int32(0), unroll=4)
                return jnp.int32(0)

            branch = jnp.where(k == 0, 0, jnp.where(rem3 == n3, 1, 2))
            lax.switch(branch, (mask_zero, mask_ge, mask_tie))
            pltpu.sync_copy(mask_v, out_hbm.at[r])
            return jnp.int32(0)

        lax.fori_loop(0, rows_per_w, do_row, jnp.int32(0))

    return body(prior_bits, ks)


def kernel(prior, t):
    rates = 1.0 - jnp.cos(jnp.pi * t / 2.0)
    weights = 0.5 * jnp.pi * jnp.sin(jnp.pi * t / 2.0)
    ks = (prior.shape[-1] * rates).astype(jnp.int32).reshape(-1)
    prior_bits = lax.bitcast_convert_type(prior, jnp.int32)
    mask = _sc_topk_mask(prior_bits, ks).astype(bool)
    return mask, weights


# speedup vs baseline: 208.2119x; 1.0640x over previous
import functools

import jax
import jax.numpy as jnp
from jax import lax
from jax.experimental import pallas as pl
from jax.experimental.pallas import tpu as pltpu
from jax.experimental.pallas import tpu_sc as plsc

_L = 16
_NB = 1024


def _iota():
    return lax.iota(jnp.int32, _L)


def _scalar(x):
    return x[0]


def _popcount(m):
    return _scalar(plsc.all_reduce_population_count(m))


def _hist_select(hist_ref, target, total):
    target_asc = total - target + 1

    def scan_vregs(j, carry):
        cum, found, jc, cum_before = carry
        s = jnp.sum(hist_ref[pl.ds(j * _L, _L)])
        crossing = jnp.logical_and(found == 0, cum + s >= target_asc)
        jc = jnp.where(crossing, j, jc)
        cum_before = jnp.where(crossing, cum, cum_before)
        found = jnp.where(crossing, jnp.int32(1), found)
        return cum + s, found, jc, cum_before

    zero = jnp.int32(0)
    _, _, jc, cum_before = lax.fori_loop(
        0, _NB // _L, scan_vregs, (zero, zero, zero, zero), unroll=4)
    h = hist_ref[pl.ds(jc * _L, _L)]
    pref = plsc.cumsum(h)
    lane = _scalar(plsc.all_reduce_ffs(cum_before + pref >= target_asc))
    bucket = jc * _L + lane
    cum_at_b = cum_before + jnp.sum(jnp.where(_iota() <= lane, h, 0))
    rem = target - (total - cum_at_b)
    return bucket, rem


def _sc_topk_mask(prior_bits, ks):
    n_rows, s = prior_bits.shape
    info = plsc.get_sparse_core_info()
    nw = info.num_cores * info.num_subcores
    rows_per_w = n_rows // nw
    nv_row = s // _L
    mesh = plsc.VectorSubcoreMesh(core_axis_name="c", subcore_axis_name="s")

    @functools.partial(
        pl.kernel,
        mesh=mesh,
        out_type=jax.ShapeDtypeStruct((n_rows, s), jnp.int32),
        scratch_types=[
            pltpu.VMEM((s,), jnp.int32),
            pltpu.VMEM((s,), jnp.int32),
            pltpu.VMEM((s,), jnp.int32),
            pltpu.VMEM((_NB + _L,), jnp.int32),
            pltpu.VMEM((n_rows + _L,), jnp.int32),
            pltpu.SemaphoreType.DMA,
            pltpu.SemaphoreType.DMA,
        ],
        compiler_params=pltpu.CompilerParams(needs_layout_passes=False),
    )
    def body(prior_hbm, ks_hbm, out_hbm, row_a, row_b, mask_v, hist_v, ks_v,
             in_sem, out_sem):
        wid = lax.axis_index("s") * info.num_cores + lax.axis_index("c")
        pltpu.sync_copy(ks_hbm, ks_v.at[pl.ds(0, n_rows)])
        ones = jnp.ones((_L,), jnp.int32)

        def zero_hist():
            zeros = jnp.zeros((_L,), jnp.int32)
            for j in range(_NB // _L):
                hist_v[pl.ds(j * _L, _L)] = zeros

        def hist_at(b):
            return _scalar(hist_v[pl.ds(b, _L)])

        def do_row(i, row_v, in_copy, out_copy):
            r = wid * rows_per_w + i
            in_copy.wait()
            next_copy = None
            if i + 1 < rows_per_w:
                next_copy = pltpu.async_copy(
                    prior_hbm.at[r + 1],
                    row_b if i % 2 == 0 else row_a, in_sem)
            k = ks_v[pl.ds(r, _L)][0]
            kk = jnp.maximum(k, 1)

            zero_hist()

            @plsc.parallel_loop(0, nv_row, 1, unroll=8)
            def _(j):
                u = row_v[pl.ds(j * _L, _L)]
                plsc.addupdate_scatter(hist_v, [u >> 20], ones)

            b1, rem1 = _hist_select(hist_v, kk, jnp.int32(s))
            n1 = hist_at(b1)

            zero_hist()

            @plsc.parallel_loop(0, nv_row, 1, unroll=8)
            def _(j):
                u = row_v[pl.ds(j * _L, _L)]
                m2 = (u >> 20) == b1
                plsc.addupdate_scatter(
                    hist_v, [(u >> 10) & (_NB - 1)], ones, mask=m2)

            b2, rem2 = _hist_select(hist_v, rem1, n1)
            n2 = hist_at(b2)
            hi20 = (b1 << 10) | b2

            zero_hist()

            @plsc.parallel_loop(0, nv_row, 1, unroll=8)
            def _(j):
                u = row_v[pl.ds(j * _L, _L)]
                m3 = (u >> 10) == hi20
                plsc.addupdate_scatter(hist_v, [u & (_NB - 1)], ones, mask=m3)

            b3, rem3 = _hist_select(hist_v, rem2, n2)
            n3 = hist_at(b3)
            thresh = (hi20 << 10) | b3

            def mask_zero():
                @plsc.parallel_loop(0, nv_row, 1, unroll=8)
                def _(j):
                    mask_v[pl.ds(j * _L, _L)] = jnp.zeros((_L,), jnp.int32)
                return jnp.int32(0)

            def mask_ge():
                @plsc.parallel_loop(0, nv_row, 1, unroll=8)
                def _(j):
                    u = row_v[pl.ds(j * _L, _L)]
                    mask_v[pl.ds(j * _L, _L)] = jnp.where(u >= thresh, 1, 0)
                return jnp.int32(0)

            def mask_tie():
                def tie(j, cnt):
                    u = row_v[pl.ds(j * _L, _L)]
                    gt = u > thresh
                    eq = u == thresh
                    pref = plsc.cumsum(jnp.where(eq, 1, 0))
                    sel = jnp.logical_and(eq, cnt + pref <= rem3)
                    mask_v[pl.ds(j * _L, _L)] = jnp.where(
                        jnp.logical_or(gt, sel), 1, 0)
                    return cnt + pref[_L - 1]

                lax.fori_loop(0, nv_row, tie, jnp.int32(0), unroll=4)
                return jnp.int32(0)

            branch = jnp.where(k == 0, 0, jnp.where(rem3 == n3, 1, 2))
            if out_copy is not None:
                out_copy.wait()
            lax.switch(branch, (mask_zero, mask_ge, mask_tie))
            return next_copy, pltpu.async_copy(mask_v, out_hbm.at[r], out_sem)

        in_copy = pltpu.async_copy(
            prior_hbm.at[wid * rows_per_w], row_a, in_sem)
        out_copy = None
        for i in range(rows_per_w):
            in_copy, out_copy = do_row(
                i, row_a if i % 2 == 0 else row_b, in_copy, out_copy)
        out_copy.wait()

    return body(prior_bits, ks)


def kernel(prior, t):
    rates = 1.0 - jnp.cos(jnp.pi * t / 2.0)
    weights = 0.5 * jnp.pi * jnp.sin(jnp.pi * t / 2.0)
    ks = (prior.shape[-1] * rates).astype(jnp.int32).reshape(-1)
    prior_bits = lax.bitcast_convert_type(prior, jnp.int32)
    mask = _sc_topk_mask(prior_bits, ks).astype(bool)
    return mask, weights


# speedup vs baseline: 222.3164x; 1.0677x over previous
import functools

import jax
import jax.numpy as jnp
from jax import lax
from jax.experimental import pallas as pl
from jax.experimental.pallas import tpu as pltpu
from jax.experimental.pallas import tpu_sc as plsc

_L = 16
_NB = 1024


def _iota():
    return lax.iota(jnp.int32, _L)


def _scalar(x):
    return x[0]


def _popcount(m):
    return _scalar(plsc.all_reduce_population_count(m))


def _hist_select(hist_ref, target, total):
    target_asc = total - target + 1

    def scan_vregs(j, carry):
        cum, found, jc, cum_before = carry
        s = jnp.sum(hist_ref[pl.ds(j * _L, _L)])
        crossing = jnp.logical_and(found == 0, cum + s >= target_asc)
        jc = jnp.where(crossing, j, jc)
        cum_before = jnp.where(crossing, cum, cum_before)
        found = jnp.where(crossing, jnp.int32(1), found)
        return cum + s, found, jc, cum_before

    zero = jnp.int32(0)
    _, _, jc, cum_before = lax.fori_loop(
        0, _NB // _L, scan_vregs, (zero, zero, zero, zero), unroll=4)
    h = hist_ref[pl.ds(jc * _L, _L)]
    pref = plsc.cumsum(h)
    lane = _scalar(plsc.all_reduce_ffs(cum_before + pref >= target_asc))
    bucket = jc * _L + lane
    cum_at_b = cum_before + jnp.sum(jnp.where(_iota() <= lane, h, 0))
    rem = target - (total - cum_at_b)
    return bucket, rem


def _sc_topk_mask(prior_f32, ks):
    n_rows, s = prior_f32.shape
    info = plsc.get_sparse_core_info()
    nw = info.num_cores * info.num_subcores
    rows_per_w = n_rows // nw
    nv_row = s // _L
    mesh = plsc.VectorSubcoreMesh(core_axis_name="c", subcore_axis_name="s")

    @functools.partial(
        pl.kernel,
        mesh=mesh,
        out_type=jax.ShapeDtypeStruct((n_rows, s), jnp.int32),
        scratch_types=[
            pltpu.VMEM((s,), jnp.float32),
            pltpu.VMEM((s,), jnp.float32),
            pltpu.VMEM((s,), jnp.int32),
            pltpu.VMEM((_NB + _L,), jnp.int32),
            pltpu.VMEM((n_rows + _L,), jnp.int32),
            pltpu.SemaphoreType.DMA,
            pltpu.SemaphoreType.DMA,
        ],
        compiler_params=pltpu.CompilerParams(needs_layout_passes=False),
    )
    def body(prior_hbm, ks_hbm, out_hbm, row_a, row_b, mask_v, hist_v, ks_v,
             in_sem, out_sem):
        wid = lax.axis_index("s") * info.num_cores + lax.axis_index("c")
        pltpu.sync_copy(ks_hbm, ks_v.at[pl.ds(0, n_rows)])
        ones = jnp.ones((_L,), jnp.int32)

        def zero_hist():
            zeros = jnp.zeros((_L,), jnp.int32)
            for j in range(_NB // _L):
                hist_v[pl.ds(j * _L, _L)] = zeros

        def hist_at(b):
            return _scalar(hist_v[pl.ds(b, _L)])

        def do_row(i, row_v, in_copy, out_copy):
            r = wid * rows_per_w + i
            in_copy.wait()
            next_copy = None
            if i + 1 < rows_per_w:
                next_copy = pltpu.async_copy(
                    prior_hbm.at[r + 1],
                    row_b if i % 2 == 0 else row_a, in_sem)
            k = ks_v[pl.ds(r, _L)][0]
            kk = jnp.maximum(k, 1)

            zero_hist()

            @plsc.parallel_loop(0, nv_row, 1, unroll=8)
            def _(j):
                u = plsc.bitcast(row_v[pl.ds(j * _L, _L)], jnp.int32)
                plsc.addupdate_scatter(hist_v, [u >> 20], ones)

            b1, rem1 = _hist_select(hist_v, kk, jnp.int32(s))
            n1 = hist_at(b1)

            zero_hist()

            @plsc.parallel_loop(0, nv_row, 1, unroll=8)
            def _(j):
                u = plsc.bitcast(row_v[pl.ds(j * _L, _L)], jnp.int32)
                m2 = (u >> 20) == b1
                plsc.addupdate_scatter(
                    hist_v, [(u >> 10) & (_NB - 1)], ones, mask=m2)

            b2, rem2 = _hist_select(hist_v, rem1, n1)
            n2 = hist_at(b2)
            hi20 = (b1 << 10) | b2

            zero_hist()

            @plsc.parallel_loop(0, nv_row, 1, unroll=8)
            def _(j):
                u = plsc.bitcast(row_v[pl.ds(j * _L, _L)], jnp.int32)
                m3 = (u >> 10) == hi20
                plsc.addupdate_scatter(hist_v, [u & (_NB - 1)], ones, mask=m3)

            b3, rem3 = _hist_select(hist_v, rem2, n2)
            n3 = hist_at(b3)
            thresh = (hi20 << 10) | b3

            def mask_zero():
                @plsc.parallel_loop(0, nv_row, 1, unroll=8)
                def _(j):
                    mask_v[pl.ds(j * _L, _L)] = jnp.zeros((_L,), jnp.int32)
                return jnp.int32(0)

            def mask_ge():
                @plsc.parallel_loop(0, nv_row, 1, unroll=8)
                def _(j):
                    u = plsc.bitcast(row_v[pl.ds(j * _L, _L)], jnp.int32)
                    mask_v[pl.ds(j * _L, _L)] = jnp.where(u >= thresh, 1, 0)
                return jnp.int32(0)

            def mask_tie():
                def tie(j, cnt):
                    u = plsc.bitcast(row_v[pl.ds(j * _L, _L)], jnp.int32)
                    gt = u > thresh
                    eq = u == thresh
                    pref = plsc.cumsum(jnp.where(eq, 1, 0))
                    sel = jnp.logical_and(eq, cnt + pref <= rem3)
                    mask_v[pl.ds(j * _L, _L)] = jnp.where(
                        jnp.logical_or(gt, sel), 1, 0)
                    return cnt + pref[_L - 1]

                lax.fori_loop(0, nv_row, tie, jnp.int32(0), unroll=4)
                return jnp.int32(0)

            branch = jnp.where(k == 0, 0, jnp.where(rem3 == n3, 1, 2))
            if out_copy is not None:
                out_copy.wait()
            lax.switch(branch, (mask_zero, mask_ge, mask_tie))
            return next_copy, pltpu.async_copy(mask_v, out_hbm.at[r], out_sem)

        in_copy = pltpu.async_copy(
            prior_hbm.at[wid * rows_per_w], row_a, in_sem)
        out_copy = None
        for i in range(rows_per_w):
            in_copy, out_copy = do_row(
                i, row_a if i % 2 == 0 else row_b, in_copy, out_copy)
        out_copy.wait()

    return body(prior_f32, ks)


def kernel(prior, t):
    rates = 1.0 - jnp.cos(jnp.pi * t / 2.0)
    weights = 0.5 * jnp.pi * jnp.sin(jnp.pi * t / 2.0)
    ks = (prior.shape[-1] * rates).astype(jnp.int32).reshape(-1)
    mask = _sc_topk_mask(prior, ks).astype(bool)
    return mask, weights
